# Initial kernel scaffold; baseline (speedup 1.0000x reference)
import jax
import jax.numpy as jnp
from jax.experimental import pallas as pl


def kernel(x, e_i_ts):
    raise NotImplementedError("write your pallas kernel here")


# speedup vs baseline: 1.0373x; 1.0373x over previous
import functools

import jax
import jax.numpy as jnp
from jax import lax
from jax.experimental import pallas as pl
from jax.experimental.pallas import tpu as pltpu
from jax.experimental.pallas import tpu_sc as plsc

E_DIM = 64
N_CODES = 1024
N_TOK = 16 * 576
TOK_BLOCK = 512

SC_CORES = 2
SC_SUBCORES = 16
SC_WORKERS = SC_CORES * SC_SUBCORES
TOK_PER_WORKER = N_TOK // SC_WORKERS
IDX_CHUNK = 96
N_CHUNKS = TOK_PER_WORKER // IDX_CHUNK


def _argmin_body(x_ref, e_ref, idx_ref, acc_ref):
    xb = x_ref[...]
    eb = e_ref[...]
    s = jnp.dot(xb, eb, preferred_element_type=jnp.float32)
    x2 = jnp.sum(xb * xb, axis=1, keepdims=True)
    e2 = jnp.sum(eb * eb, axis=0, keepdims=True)
    d = x2 - 2.0 * s + e2
    m = jnp.min(d, axis=1, keepdims=True)
    ii = lax.broadcasted_iota(jnp.int32, d.shape, 1)
    idx = jnp.min(jnp.where(d == m, ii, jnp.int32(N_CODES)), axis=1)
    idx_ref[...] = idx

    @pl.when(pl.program_id(0) == 0)
    def _():
        acc_ref[0, 0] = jnp.float32(0.0)

    acc_ref[0, 0] += jnp.sum(m)


def _tc_argmin(flat_x, e_i_ts, interpret=False):
    grid = (N_TOK // TOK_BLOCK,)
    return pl.pallas_call(
        _argmin_body,
        grid=grid,
        in_specs=[
            pl.BlockSpec((TOK_BLOCK, E_DIM), lambda i: (i, 0)),
            pl.BlockSpec((E_DIM, N_CODES), lambda i: (0, 0)),
        ],
        out_specs=[
            pl.BlockSpec((TOK_BLOCK,), lambda i: (i,)),
            pl.BlockSpec((1, 1), lambda i: (0, 0), memory_space=pltpu.SMEM),
        ],
        out_shape=[
            jax.ShapeDtypeStruct((N_TOK,), jnp.int32),
            jax.ShapeDtypeStruct((1, 1), jnp.float32),
        ],
        interpret=interpret,
    )(flat_x, e_i_ts)


def _sc_gather(table, idx3d):
    mesh = plsc.VectorSubcoreMesh(core_axis_name="c", subcore_axis_name="s")

    @functools.partial(
        pl.kernel,
        mesh=mesh,
        out_type=jax.ShapeDtypeStruct((N_TOK, E_DIM), jnp.float32),
        scratch_types=[
            pltpu.VMEM((N_CHUNKS, IDX_CHUNK), jnp.int32),
            pltpu.VMEM((TOK_PER_WORKER, E_DIM), jnp.float32),
            pltpu.SemaphoreType.DMA,
        ],
        compiler_params=pltpu.CompilerParams(use_tc_tiling_on_sc=False),
    )
    def gather_kernel(table_hbm, idx_hbm, out_hbm, idx_v, rows_v, sem):
        wid = lax.axis_index("s") * SC_CORES + lax.axis_index("c")
        base = wid * TOK_PER_WORKER
        pltpu.sync_copy(idx_hbm.at[wid], idx_v)
        copies = [
            pltpu.async_copy(
                table_hbm.at[idx_v.at[j]],
                rows_v.at[pl.ds(j * IDX_CHUNK, IDX_CHUNK)],
                sem,
            )
            for j in range(N_CHUNKS)
        ]
        for c in copies:
            c.wait()
        pltpu.sync_copy(rows_v, out_hbm.at[pl.ds(base, TOK_PER_WORKER)])

    return gather_kernel(table, idx3d)


def kernel(x, e_i_ts):
    B, L, E = x.shape
    flat_x = x.reshape(B * L, E)
    idx, acc = _tc_argmin(flat_x, e_i_ts)
    table = e_i_ts.T
    flat_q = _sc_gather(table, idx.reshape(SC_WORKERS, N_CHUNKS, IDX_CHUNK))
    quantized = flat_q.reshape(B, L, E)
    loss = acc[0, 0] * jnp.float32(1.0 / (N_TOK * E_DIM))
    encoding_indices = idx.reshape(B, L)
    return (quantized, loss, loss, encoding_indices)


# speedup vs baseline: 1.1210x; 1.0807x over previous
import functools

import jax
import jax.numpy as jnp
from jax import lax
from jax.experimental import pallas as pl
from jax.experimental.pallas import tpu as pltpu
from jax.experimental.pallas import tpu_sc as plsc

E_DIM = 64
N_CODES = 1024
N_TOK = 16 * 576
TOK_BLOCK = 512

SC_CORES = 2
SC_SUBCORES = 16
SC_WORKERS = SC_CORES * SC_SUBCORES
TOK_PER_WORKER = N_TOK // SC_WORKERS
IDX_CHUNK = 96
N_CHUNKS = TOK_PER_WORKER // IDX_CHUNK


def _argmin_body(x_ref, e_ref, idx_ref, acc_ref):
    xb = x_ref[...]
    eb = e_ref[...]
    s = jnp.dot(xb, eb * -2.0, preferred_element_type=jnp.float32)
    x2 = jnp.sum(xb * xb, axis=1, keepdims=True)
    e2 = jnp.sum(eb * eb, axis=0, keepdims=True)
    d = x2 + s + e2
    m = jnp.min(d, axis=1, keepdims=True)
    ii = lax.broadcasted_iota(jnp.int32, d.shape, 1).astype(jnp.float32)
    idxf = jnp.min(jnp.where(d == m, ii, jnp.float32(N_CODES)), axis=1)
    idx_ref[...] = idxf.astype(jnp.int32)

    @pl.when(pl.program_id(0) == 0)
    def _():
        acc_ref[0, 0] = jnp.float32(0.0)

    acc_ref[0, 0] += jnp.sum(m)


def _tc_argmin(flat_x, e_i_ts, interpret=False):
    grid = (N_TOK // TOK_BLOCK,)
    return pl.pallas_call(
        _argmin_body,
        grid=grid,
        in_specs=[
            pl.BlockSpec((TOK_BLOCK, E_DIM), lambda i: (i, 0)),
            pl.BlockSpec((E_DIM, N_CODES), lambda i: (0, 0)),
        ],
        out_specs=[
            pl.BlockSpec((TOK_BLOCK,), lambda i: (i,)),
            pl.BlockSpec((1, 1), lambda i: (0, 0), memory_space=pltpu.SMEM),
        ],
        out_shape=[
            jax.ShapeDtypeStruct((N_TOK,), jnp.int32),
            jax.ShapeDtypeStruct((1, 1), jnp.float32),
        ],
        interpret=interpret,
    )(flat_x, e_i_ts)


def _sc_gather(table, idx3d):
    mesh = plsc.VectorSubcoreMesh(core_axis_name="c", subcore_axis_name="s")

    @functools.partial(
        pl.kernel,
        mesh=mesh,
        out_type=jax.ShapeDtypeStruct((N_TOK, E_DIM), jnp.float32),
        scratch_types=[
            pltpu.VMEM((N_CHUNKS, IDX_CHUNK), jnp.int32),
            pltpu.VMEM((TOK_PER_WORKER, E_DIM), jnp.float32),
            pltpu.SemaphoreType.DMA,
        ],
        compiler_params=pltpu.CompilerParams(use_tc_tiling_on_sc=False),
    )
    def gather_kernel(table_hbm, idx_hbm, out_hbm, idx_v, rows_v, sem):
        wid = lax.axis_index("s") * SC_CORES + lax.axis_index("c")
        base = wid * TOK_PER_WORKER
        pltpu.sync_copy(idx_hbm.at[wid], idx_v)
        copies = [
            pltpu.async_copy(
                table_hbm.at[idx_v.at[j]],
                rows_v.at[pl.ds(j * IDX_CHUNK, IDX_CHUNK)],
                sem,
            )
            for j in range(N_CHUNKS)
        ]
        for c in copies:
            c.wait()
        pltpu.sync_copy(rows_v, out_hbm.at[pl.ds(base, TOK_PER_WORKER)])

    return gather_kernel(table, idx3d)


def kernel(x, e_i_ts):
    B, L, E = x.shape
    flat_x = x.reshape(B * L, E)
    idx, acc = _tc_argmin(flat_x, e_i_ts)
    table = e_i_ts.T
    flat_q = _sc_gather(table, idx.reshape(SC_WORKERS, N_CHUNKS, IDX_CHUNK))
    quantized = flat_q.reshape(B, L, E)
    loss = acc[0, 0] * jnp.float32(1.0 / (N_TOK * E_DIM))
    encoding_indices = idx.reshape(B, L)
    return (quantized, loss, loss, encoding_indices)
